# Initial kernel scaffold; baseline (speedup 1.0000x reference)
import jax
import jax.numpy as jnp
from jax.experimental import pallas as pl


def kernel(x, edge_index, W1, al1, ar1, W2, al2, ar2):
    raise NotImplementedError("write your pallas kernel here")


# speedup vs baseline: 3.1359x; 3.1359x over previous
import functools

import jax
import jax.numpy as jnp
from jax import lax
from jax.experimental import pallas as pl
from jax.experimental.pallas import tpu as pltpu
from jax.experimental.pallas import tpu_sc as plsc

N = 10000
E = 160000
F_IN = 256
HID = 256
HEADS = 8
NCLS = 40
NPIECE = 16
E_PAD = 163840
NB = 400
EPS = 1e-30

NW = 32
EW = E_PAD // NW
EROWS = EW // 128
NA = 10240
NROWT = NA // 16
ETILE = E_PAD // 16
ESROWS = 16

f32 = jnp.float32
i32 = jnp.int32

_sc_mesh = plsc.VectorSubcoreMesh(core_axis_name="c", subcore_axis_name="s")
_sc_params = pltpu.CompilerParams(needs_layout_passes=False)


def _build_dbig(al1, ar1):
    H, F = al1.shape
    d = jnp.zeros((H * F, 128), f32)
    for h in range(H):
        for r in (2 * h, 2 * h + 1):
            d = d.at[h * F:(h + 1) * F, r].set(al1[h])
            d = d.at[h * F:(h + 1) * F, 16 + r].set(ar1[h])
    return d


def _tca_body(x_ref, w1_ref, dbig_ref, fg_ref, t_ref):
    f = jnp.dot(x_ref[...], w1_ref[...], preferred_element_type=f32)
    fg_ref[...] = f.reshape(NB, NPIECE, 128).transpose(1, 0, 2)
    t_ref[...] = jnp.dot(f, dbig_ref[...], preferred_element_type=f32)


def _tca(x, W1, d_big):
    return pl.pallas_call(
        _tca_body,
        grid=(N // NB,),
        in_specs=[
            pl.BlockSpec((NB, F_IN), lambda i: (i, 0)),
            pl.BlockSpec((F_IN, HEADS * HID), lambda i: (0, 0)),
            pl.BlockSpec((HEADS * HID, 128), lambda i: (0, 0)),
        ],
        out_specs=[
            pl.BlockSpec((NPIECE, NB, 128), lambda i: (0, i, 0)),
            pl.BlockSpec((NB, 128), lambda i: (i, 0)),
        ],
        out_shape=[
            jax.ShapeDtypeStruct((NPIECE, N, 128), f32),
            jax.ShapeDtypeStruct((N, 128), f32),
        ],
    )(x, W1, d_big)


def _scb_body(t_hbm, src2_hbm, dst2_hbm, w1e_out,
              idx_s, idx_d, elrows, errows, wbuf16):
    c = lax.axis_index("c")
    s = lax.axis_index("s")
    wid = s * 2 + c
    rowbase = wid * EROWS
    ebase = wid * EW
    pltpu.sync_copy(src2_hbm.at[pl.ds(rowbase, EROWS), :], idx_s)
    pltpu.sync_copy(dst2_hbm.at[pl.ds(rowbase, EROWS), :], idx_d)

    def _chunk(k, _):
        pltpu.sync_copy(t_hbm.at[idx_s.at[k]], elrows)
        pltpu.sync_copy(t_hbm.at[idx_d.at[k]], errows)

        def _edge(j, _):
            el = elrows[j, pl.ds(0, 16)]
            er = errows[j, pl.ds(16, 16)]
            e = el + er
            e = jnp.where(e > 0, e, 0.2 * e)
            w = jnp.exp(e)
            eid = ebase + k * 128 + j
            w = jnp.where(jnp.full((16,), eid, i32) < E, w, 0.0)
            wbuf16[j, :] = w
            return 0
        lax.fori_loop(0, 128, _edge, 0)
        pltpu.sync_copy(wbuf16, w1e_out.at[pl.ds(ebase + k * 128, 128), :])
        return 0
    lax.fori_loop(0, EROWS, _chunk, 0)


_scb = functools.partial(
    pl.kernel,
    out_type=jax.ShapeDtypeStruct((E_PAD, NPIECE), f32),
    mesh=_sc_mesh,
    compiler_params=_sc_params,
    scratch_types=[
        pltpu.VMEM((EROWS, 128), i32),
        pltpu.VMEM((EROWS, 128), i32),
        pltpu.VMEM((128, 128), f32),
        pltpu.VMEM((128, 128), f32),
        pltpu.VMEM((128, NPIECE), f32),
    ],
)(_scb_body)


def _scc_body(ff_hbm, w1e_hbm, src2_hbm, dst2_hbm, out1p, den1p,
              idx_s, idx_d, qbuf, rowsa, rowsb, wchunk, acc_sh):
    c = lax.axis_index("c")
    s = lax.axis_index("s")

    def _zero_acc():
        def _zrow(j, _):
            for t in range(8):
                rowsb[j, pl.ds(t * 16, 16)] = jnp.zeros((16,), f32)
            return 0
        lax.fori_loop(0, 128, _zrow, 0)
        for i in range(5):
            r0 = s * NROWT + i * 128
            pltpu.sync_copy(rowsb, acc_sh.at[pl.ds(r0, 128), :])

    def _copy_out(dst):
        for i in range(5):
            r0 = s * NROWT + i * 128
            pltpu.sync_copy(acc_sh.at[pl.ds(r0, 128), :], rowsb)
            pltpu.sync_copy(rowsb, dst.at[pl.ds(r0, 128), :])

    def _pass(p, _):
        q = c * 8 + p
        _zero_acc()
        plsc.subcore_barrier()

        def _stage(stage, _):
            rb = s * 80 + stage * ESROWS
            pltpu.sync_copy(src2_hbm.at[pl.ds(rb, ESROWS), :], idx_s)
            pltpu.sync_copy(dst2_hbm.at[pl.ds(rb, ESROWS), :], idx_d)

            def _chunk(k, _):
                for t in range(8):
                    qbuf[pl.ds(t * 16, 16)] = (
                        idx_s[k, pl.ds(t * 16, 16)] + q * N)
                pltpu.sync_copy(
                    w1e_hbm.at[pl.ds(rb * 128 + k * 128, 128), :], wchunk)
                for hh in range(2):
                    pltpu.sync_copy(
                        ff_hbm.at[qbuf.at[pl.ds(hh * 64, 64)]], rowsa)

                    def _edge(j, _):
                        ws = plsc.load_gather(
                            wchunk,
                            [jnp.full((16,), hh * 64, i32) + j,
                             jnp.full((16,), q, i32)])
                        for t in range(8):
                            r = rowsa[j, pl.ds(t * 16, 16)]
                            rowsb[hh * 64 + j, pl.ds(t * 16, 16)] = r * ws
                        return 0
                    lax.fori_loop(0, 64, _edge, 0)
                pltpu.sync_copy(rowsb, acc_sh.at[idx_d.at[k]], add=True)
                return 0
            lax.fori_loop(0, ESROWS, _chunk, 0)
            return 0
        lax.fori_loop(0, 5, _stage, 0)
        plsc.subcore_barrier()
        _copy_out(out1p.at[q])
        plsc.subcore_barrier()
        return 0
    lax.fori_loop(0, 8, _pass, 0)

    _zero_acc()
    plsc.subcore_barrier()

    def _dstage(stage, _):
        rb = c * 640 + s * 40 + stage * 8
        pltpu.sync_copy(dst2_hbm.at[pl.ds(rb, 8), :],
                        idx_d.at[pl.ds(0, 8), :])

        def _dchunk(k, _):
            pltpu.sync_copy(
                w1e_hbm.at[pl.ds(rb * 128 + k * 128, 128), :], wchunk)

            def _edge(j, _):
                rowsb[j, pl.ds(0, 16)] = wchunk[j, :]
                return 0
            lax.fori_loop(0, 128, _edge, 0)
            pltpu.sync_copy(rowsb, acc_sh.at[idx_d.at[k]], add=True)
            return 0
        lax.fori_loop(0, 8, _dchunk, 0)
        return 0
    lax.fori_loop(0, 5, _dstage, 0)
    plsc.subcore_barrier()
    _copy_out(den1p.at[c])


_scc = functools.partial(
    pl.kernel,
    out_type=[
        jax.ShapeDtypeStruct((NPIECE, NA, 128), f32),
        jax.ShapeDtypeStruct((2, NA, 128), f32),
    ],
    mesh=_sc_mesh,
    compiler_params=_sc_params,
    scratch_types=[
        pltpu.VMEM((ESROWS, 128), i32),
        pltpu.VMEM((ESROWS, 128), i32),
        pltpu.VMEM((128,), i32),
        pltpu.VMEM((64, 128), f32),
        pltpu.VMEM((128, 128), f32),
        pltpu.VMEM((128, NPIECE), f32),
        pltpu.VMEM_SHARED((NA, 128), f32),
    ],
)(_scc_body)


def _tcd_body(og_ref, den_ref, w2_ref, j_ref, f2_ref, elr_ref):
    den = den_ref[0, :, :NPIECE] + den_ref[1, :, :NPIECE] + EPS
    h = og_ref[...].transpose(1, 0, 2) / den[:, :, None]
    h = jax.nn.relu(h.reshape(NB, HEADS * HID))
    f2 = jnp.dot(h, w2_ref[...], preferred_element_type=f32)
    ones_col = lax.broadcasted_iota(i32, (NB, 128), 1) == 47
    f2 = jnp.where(ones_col, 1.0, f2)
    f2_ref[...] = f2
    elr_ref[...] = jnp.dot(f2, j_ref[...], preferred_element_type=f32)


def _tcd(out1p, den1p, W2p, Jmat):
    return pl.pallas_call(
        _tcd_body,
        grid=(N // NB,),
        in_specs=[
            pl.BlockSpec((NPIECE, NB, 128), lambda i: (0, i, 0)),
            pl.BlockSpec((2, NB, 128), lambda i: (0, i, 0)),
            pl.BlockSpec((HEADS * HID, 128), lambda i: (0, 0)),
            pl.BlockSpec((128, 8), lambda i: (0, 0)),
        ],
        out_specs=[
            pl.BlockSpec((NB, 128), lambda i: (i, 0)),
            pl.BlockSpec((NB, 8), lambda i: (i, 0)),
        ],
        out_shape=[
            jax.ShapeDtypeStruct((N, 128), f32),
            jax.ShapeDtypeStruct((N, 8), f32),
        ],
    )(out1p, den1p, W2p, Jmat)


def _sce_body(src2_hbm, dst2_hbm, f2p_hbm, el2_hbm, er2_hbm, acc_out,
              idx_s, idx_d, el2v, er2v, rowsa, rowsb, w2b, idx2, acc_sh):
    c = lax.axis_index("c")
    s = lax.axis_index("s")
    wid = s * 2 + c
    rowbase = wid * EROWS
    ebase = wid * EW
    pltpu.sync_copy(src2_hbm.at[pl.ds(rowbase, EROWS), :], idx_s)
    pltpu.sync_copy(dst2_hbm.at[pl.ds(rowbase, EROWS), :], idx_d)
    pltpu.sync_copy(el2_hbm, el2v)
    pltpu.sync_copy(er2_hbm, er2v)

    for h in range(2):
        lo = h * (NA // 2)

        def _zrow(j, _):
            for t in range(8):
                rowsb[j, pl.ds(t * 16, 16)] = jnp.zeros((16,), f32)
            return 0
        lax.fori_loop(0, 128, _zrow, 0)
        for off, sz in ((0, 128), (128, 128), (256, 64)):
            pltpu.sync_copy(rowsb.at[pl.ds(0, sz), :],
                            acc_sh.at[pl.ds(s * 320 + off, sz), :])
        plsc.subcore_barrier()

        def _chunk(k, _):
            pltpu.sync_copy(f2p_hbm.at[idx_s.at[k]], rowsa)
            for t in range(8):
                sv = idx_s[k, pl.ds(t * 16, 16)]
                dv = idx_d[k, pl.ds(t * 16, 16)]
                elv = plsc.load_gather(el2v, [sv])
                erv = plsc.load_gather(er2v, [dv])
                e = elv + erv
                e = jnp.where(e > 0, e, 0.2 * e)
                w = jnp.exp(e)
                eid = ebase + k * 128 + t * 16 + lax.iota(i32, 16)
                w = jnp.where(eid < E, w, 0.0)
                inh = (dv >= lo) & (dv < lo + NA // 2)
                w = jnp.where(inh, w, 0.0)
                w2b[pl.ds(t * 16, 16)] = w
                iv = jnp.clip(dv - lo, 0, NA // 2 - 1)
                idx2[0, pl.ds(t * 16, 16)] = iv

            def _scale(j, _):
                ws = plsc.load_gather(w2b, [jnp.full((16,), j, i32)])
                for t in range(8):
                    r = rowsa[j, pl.ds(t * 16, 16)]
                    rowsb[j, pl.ds(t * 16, 16)] = r * ws
                return 0
            lax.fori_loop(0, 128, _scale, 0)
            pltpu.sync_copy(rowsb, acc_sh.at[idx2.at[0]], add=True)
            return 0
        lax.fori_loop(0, EROWS, _chunk, 0)
        plsc.subcore_barrier()

        for off, sz in ((0, 128), (128, 128), (256, 64)):
            pltpu.sync_copy(acc_sh.at[pl.ds(s * 320 + off, sz), :],
                            rowsa.at[pl.ds(0, sz), :])
            pltpu.sync_copy(
                rowsa.at[pl.ds(0, sz), :],
                acc_out.at[c, pl.ds(lo + s * 320 + off, sz), :])
        plsc.subcore_barrier()


_sce = functools.partial(
    pl.kernel,
    out_type=jax.ShapeDtypeStruct((2, NA, 128), f32),
    mesh=_sc_mesh,
    compiler_params=_sc_params,
    scratch_types=[
        pltpu.VMEM((EROWS, 128), i32),
        pltpu.VMEM((EROWS, 128), i32),
        pltpu.VMEM((NA,), f32),
        pltpu.VMEM((NA,), f32),
        pltpu.VMEM((128, 128), f32),
        pltpu.VMEM((128, 128), f32),
        pltpu.VMEM((128,), f32),
        pltpu.VMEM((1, 128), i32),
        pltpu.VMEM_SHARED((NA // 2, 128), f32),
    ],
)(_sce_body)


def _tcf_body(acc_ref, out_ref):
    a = acc_ref[0] + acc_ref[1]
    out_ref[...] = a[:, :NCLS] / (a[:, 47:48] + EPS)


def _tcf(acc2p):
    return pl.pallas_call(
        _tcf_body,
        grid=(N // NB,),
        in_specs=[pl.BlockSpec((2, NB, 128), lambda i: (0, i, 0))],
        out_specs=pl.BlockSpec((NB, NCLS), lambda i: (i, 0)),
        out_shape=jax.ShapeDtypeStruct((N, NCLS), f32),
    )(acc2p)


_USE_SC_C = True
_USE_SC_E = True


def kernel(x, edge_index, W1, al1, ar1, W2, al2, ar2):
    src = edge_index[0]
    dst = edge_index[1]
    pad = E_PAD - E
    srcp = jnp.concatenate([src, jnp.zeros((pad,), i32)])
    dstp = jnp.concatenate([dst, jnp.zeros((pad,), i32)])
    src2 = srcp.reshape(E_PAD // 128, 128)
    dst2 = dstp.reshape(E_PAD // 128, 128)

    d_big = _build_dbig(al1, ar1)
    W2p = jnp.zeros((HEADS * HID, 128), f32).at[:, :NCLS].set(W2)
    al2p = jnp.zeros((128,), f32).at[:NCLS].set(al2[0])
    ar2p = jnp.zeros((128,), f32).at[:NCLS].set(ar2[0])
    Jmat = jnp.zeros((128, 8), f32).at[:, 0].set(al2p).at[:, 1].set(ar2p)

    feat1_g, T1 = _tca(x, W1, d_big)
    feat1_flat = feat1_g.reshape(NPIECE * N, 128)

    w1e = _scb(T1, src2, dst2)

    out1p, den1p = _scc(feat1_flat, w1e, src2, dst2)

    feat2p, elr2 = _tcd(out1p, den1p, W2p, Jmat)
    el2 = jnp.zeros((NA,), f32).at[:N].set(elr2[:, 0])
    er2 = jnp.zeros((NA,), f32).at[:N].set(elr2[:, 1])

    acc2p = _sce(src2, dst2, feat2p, el2, er2)

    return _tcf(acc2p)


# speedup vs baseline: 4.1594x; 1.3264x over previous
import functools

import jax
import jax.numpy as jnp
from jax import lax
from jax.experimental import pallas as pl
from jax.experimental.pallas import tpu as pltpu
from jax.experimental.pallas import tpu_sc as plsc

N = 10000
E = 160000
F_IN = 256
HID = 256
HEADS = 8
NCLS = 40
NPIECE = 16
E_PAD = 163840
NB = 400
EPS = 1e-30

NW = 32
EW = E_PAD // NW
EROWS = EW // 128
NA = 10240
NROWT = NA // 16
ETILE = E_PAD // 16
ESROWS = 16

f32 = jnp.float32
i32 = jnp.int32

_sc_mesh = plsc.VectorSubcoreMesh(core_axis_name="c", subcore_axis_name="s")
_sc_params = pltpu.CompilerParams(needs_layout_passes=False)


def _build_dbig(al1, ar1):
    H, F = al1.shape
    d = jnp.zeros((H * F, 128), f32)
    for h in range(H):
        for r in (2 * h, 2 * h + 1):
            d = d.at[h * F:(h + 1) * F, r].set(al1[h])
            d = d.at[h * F:(h + 1) * F, 16 + r].set(ar1[h])
    return d


def _tca_body(x_ref, w1_ref, dbig_ref, fg_ref, t_ref):
    f = jnp.dot(x_ref[...], w1_ref[...], preferred_element_type=f32)
    fg_ref[...] = f.reshape(NB, NPIECE, 128).transpose(1, 0, 2)
    t_ref[...] = jnp.dot(f, dbig_ref[...], preferred_element_type=f32)


def _tca(x, W1, d_big):
    return pl.pallas_call(
        _tca_body,
        grid=(N // NB,),
        in_specs=[
            pl.BlockSpec((NB, F_IN), lambda i: (i, 0)),
            pl.BlockSpec((F_IN, HEADS * HID), lambda i: (0, 0)),
            pl.BlockSpec((HEADS * HID, 128), lambda i: (0, 0)),
        ],
        out_specs=[
            pl.BlockSpec((NPIECE, NB, 128), lambda i: (0, i, 0)),
            pl.BlockSpec((NB, 128), lambda i: (i, 0)),
        ],
        out_shape=[
            jax.ShapeDtypeStruct((NPIECE, N, 128), f32),
            jax.ShapeDtypeStruct((N, 128), f32),
        ],
    )(x, W1, d_big)


def _scb_body(t_hbm, src2_hbm, dst2_hbm, w1e_out,
              idx_s, idx_d, elrows, errows, wbuf16):
    c = lax.axis_index("c")
    s = lax.axis_index("s")
    wid = s * 2 + c
    rowbase = wid * EROWS
    ebase = wid * EW
    pltpu.sync_copy(src2_hbm.at[pl.ds(rowbase, EROWS), :], idx_s)
    pltpu.sync_copy(dst2_hbm.at[pl.ds(rowbase, EROWS), :], idx_d)

    def _chunk(k, _):
        pltpu.sync_copy(t_hbm.at[idx_s.at[k]], elrows)
        pltpu.sync_copy(t_hbm.at[idx_d.at[k]], errows)

        def _edge(j, _):
            el = elrows[j, pl.ds(0, 16)]
            er = errows[j, pl.ds(16, 16)]
            e = el + er
            e = jnp.where(e > 0, e, 0.2 * e)
            w = jnp.exp(e)
            eid = ebase + k * 128 + j
            w = jnp.where(jnp.full((16,), eid, i32) < E, w, 0.0)
            wbuf16[j, :] = w
            return 0
        lax.fori_loop(0, 128, _edge, 0)
        pltpu.sync_copy(wbuf16, w1e_out.at[pl.ds(ebase + k * 128, 128), :])
        return 0
    lax.fori_loop(0, EROWS, _chunk, 0)


_scb = functools.partial(
    pl.kernel,
    out_type=jax.ShapeDtypeStruct((E_PAD, NPIECE), f32),
    mesh=_sc_mesh,
    compiler_params=_sc_params,
    scratch_types=[
        pltpu.VMEM((EROWS, 128), i32),
        pltpu.VMEM((EROWS, 128), i32),
        pltpu.VMEM((128, 128), f32),
        pltpu.VMEM((128, 128), f32),
        pltpu.VMEM((128, NPIECE), f32),
    ],
)(_scb_body)


def _scc_body(ff_hbm, w1e_hbm, src2_hbm, dst2_hbm, out1p, den1p,
              idx_s, idx_d, qbuf, rowsa0, rowsa1, rowsb, wchunk,
              sem_w, sem_g0, sem_g1, sem_s, acc_sh):
    c = lax.axis_index("c")
    s = lax.axis_index("s")

    def _zero_acc():
        def _zrow(j, _):
            for t in range(8):
                rowsb[j, pl.ds(t * 16, 16)] = jnp.zeros((16,), f32)
            return 0
        lax.fori_loop(0, 128, _zrow, 0)
        for i in range(5):
            r0 = s * NROWT + i * 128
            pltpu.sync_copy(rowsb, acc_sh.at[pl.ds(r0, 128), :])

    def _copy_out(dst):
        for i in range(5):
            r0 = s * NROWT + i * 128
            pltpu.sync_copy(acc_sh.at[pl.ds(r0, 128), :], rowsb)
            pltpu.sync_copy(rowsb, dst.at[pl.ds(r0, 128), :])

    def _pass(p, _):
        q = c * 8 + p
        _zero_acc()
        plsc.subcore_barrier()

        def _stage(stage, _):
            rb = s * 80 + stage * 8
            pltpu.sync_copy(src2_hbm.at[pl.ds(rb, 8), :], idx_s)
            pltpu.sync_copy(dst2_hbm.at[pl.ds(rb, 8), :], idx_d)

            def _chunk(k, _):
                for t in range(8):
                    qbuf[pl.ds(t * 16, 16)] = (
                        idx_s[k, pl.ds(t * 16, 16)] + q * N)
                cw = pltpu.async_copy(
                    w1e_hbm.at[pl.ds(rb * 128 + k * 128, 128), :], wchunk,
                    sem_w)
                ras = (rowsa0, rowsa1)
                sems = (sem_g0, sem_g1)
                cps = [pltpu.async_copy(
                    ff_hbm.at[qbuf.at[pl.ds(0, 32)]], rowsa0, sem_g0)]

                @pl.when(jnp.logical_or(k > 0, stage > 0))
                def _():
                    pltpu.make_async_copy(
                        w1e_hbm.at[pl.ds(0, 128), :], rowsb, sem_s).wait()

                cw.wait()
                for i in range(4):
                    if i < 3:
                        cps.append(pltpu.async_copy(
                            ff_hbm.at[qbuf.at[pl.ds(32 * (i + 1), 32)]],
                            ras[(i + 1) % 2], sems[(i + 1) % 2]))
                    cps[i].wait()
                    ra = ras[i % 2]

                    def _edge(j, _):
                        ws = plsc.load_gather(
                            wchunk,
                            [jnp.full((16,), 32 * i, i32) + j,
                             jnp.full((16,), q, i32)])
                        for t in range(8):
                            r = ra[j, pl.ds(t * 16, 16)]
                            rowsb[32 * i + j, pl.ds(t * 16, 16)] = r * ws
                        return 0
                    lax.fori_loop(0, 32, _edge, 0)
                pltpu.async_copy(rowsb, acc_sh.at[idx_d.at[k]], sem_s,
                                 add=True)
                return 0
            lax.fori_loop(0, 8, _chunk, 0)
            return 0
        lax.fori_loop(0, 10, _stage, 0)
        pltpu.make_async_copy(
            w1e_hbm.at[pl.ds(0, 128), :], rowsb, sem_s).wait()
        plsc.subcore_barrier()
        _copy_out(out1p.at[q])
        plsc.subcore_barrier()
        return 0
    lax.fori_loop(0, 8, _pass, 0)

    _zero_acc()
    plsc.subcore_barrier()

    def _dstage(stage, _):
        rb = c * 640 + s * 40 + stage * 8
        pltpu.sync_copy(dst2_hbm.at[pl.ds(rb, 8), :], idx_d)

        def _dchunk(k, _):
            pltpu.sync_copy(
                w1e_hbm.at[pl.ds(rb * 128 + k * 128, 128), :], wchunk)

            def _edge(j, _):
                rowsb[j, pl.ds(0, 16)] = wchunk[j, :]
                return 0
            lax.fori_loop(0, 128, _edge, 0)
            pltpu.sync_copy(rowsb, acc_sh.at[idx_d.at[k]], add=True)
            return 0
        lax.fori_loop(0, 8, _dchunk, 0)
        return 0
    lax.fori_loop(0, 5, _dstage, 0)
    plsc.subcore_barrier()
    _copy_out(den1p.at[c])


_scc = functools.partial(
    pl.kernel,
    out_type=[
        jax.ShapeDtypeStruct((NPIECE, NA, 128), f32),
        jax.ShapeDtypeStruct((2, NA, 128), f32),
    ],
    mesh=_sc_mesh,
    compiler_params=_sc_params,
    scratch_types=[
        pltpu.VMEM((8, 128), i32),
        pltpu.VMEM((8, 128), i32),
        pltpu.VMEM((128,), i32),
        pltpu.VMEM((32, 128), f32),
        pltpu.VMEM((32, 128), f32),
        pltpu.VMEM((128, 128), f32),
        pltpu.VMEM((128, NPIECE), f32),
        pltpu.SemaphoreType.DMA,
        pltpu.SemaphoreType.DMA,
        pltpu.SemaphoreType.DMA,
        pltpu.SemaphoreType.DMA,
        pltpu.VMEM_SHARED((NA, 128), f32),
    ],
)(_scc_body)


def _tcd_body(og_ref, den_ref, w2_ref, j_ref, f2_ref, elr_ref):
    den = den_ref[0, :, :NPIECE] + den_ref[1, :, :NPIECE] + EPS
    h = og_ref[...].transpose(1, 0, 2) / den[:, :, None]
    h = jax.nn.relu(h.reshape(NB, HEADS * HID))
    f2 = jnp.dot(h, w2_ref[...], preferred_element_type=f32)
    ones_col = lax.broadcasted_iota(i32, (NB, 128), 1) == 47
    f2 = jnp.where(ones_col, 1.0, f2)
    f2_ref[...] = f2
    elr_ref[...] = jnp.dot(f2, j_ref[...], preferred_element_type=f32)


def _tcd(out1p, den1p, W2p, Jmat):
    return pl.pallas_call(
        _tcd_body,
        grid=(N // NB,),
        in_specs=[
            pl.BlockSpec((NPIECE, NB, 128), lambda i: (0, i, 0)),
            pl.BlockSpec((2, NB, 128), lambda i: (0, i, 0)),
            pl.BlockSpec((HEADS * HID, 128), lambda i: (0, 0)),
            pl.BlockSpec((128, 8), lambda i: (0, 0)),
        ],
        out_specs=[
            pl.BlockSpec((NB, 128), lambda i: (i, 0)),
            pl.BlockSpec((NB, 8), lambda i: (i, 0)),
        ],
        out_shape=[
            jax.ShapeDtypeStruct((N, 128), f32),
            jax.ShapeDtypeStruct((N, 8), f32),
        ],
    )(out1p, den1p, W2p, Jmat)


def _sce_body(src2_hbm, dst2_hbm, f2p_hbm, el2_hbm, er2_hbm, acc_out,
              idx_s, idx_d, el2v, er2v, rowsa, rowsb, w2b, idx2, acc_sh):
    c = lax.axis_index("c")
    s = lax.axis_index("s")
    wid = s * 2 + c
    rowbase = wid * EROWS
    ebase = wid * EW
    pltpu.sync_copy(src2_hbm.at[pl.ds(rowbase, EROWS), :], idx_s)
    pltpu.sync_copy(dst2_hbm.at[pl.ds(rowbase, EROWS), :], idx_d)
    pltpu.sync_copy(el2_hbm, el2v)
    pltpu.sync_copy(er2_hbm, er2v)

    for h in range(2):
        lo = h * (NA // 2)

        def _zrow(j, _):
            for t in range(8):
                rowsb[j, pl.ds(t * 16, 16)] = jnp.zeros((16,), f32)
            return 0
        lax.fori_loop(0, 128, _zrow, 0)
        for off, sz in ((0, 128), (128, 128), (256, 64)):
            pltpu.sync_copy(rowsb.at[pl.ds(0, sz), :],
                            acc_sh.at[pl.ds(s * 320 + off, sz), :])
        plsc.subcore_barrier()

        def _chunk(k, _):
            pltpu.sync_copy(f2p_hbm.at[idx_s.at[k]], rowsa)
            for t in range(8):
                sv = idx_s[k, pl.ds(t * 16, 16)]
                dv = idx_d[k, pl.ds(t * 16, 16)]
                elv = plsc.load_gather(el2v, [sv])
                erv = plsc.load_gather(er2v, [dv])
                e = elv + erv
                e = jnp.where(e > 0, e, 0.2 * e)
                w = jnp.exp(e)
                eid = ebase + k * 128 + t * 16 + lax.iota(i32, 16)
                w = jnp.where(eid < E, w, 0.0)
                inh = (dv >= lo) & (dv < lo + NA // 2)
                w = jnp.where(inh, w, 0.0)
                w2b[pl.ds(t * 16, 16)] = w
                iv = jnp.clip(dv - lo, 0, NA // 2 - 1)
                idx2[0, pl.ds(t * 16, 16)] = iv

            def _scale(j, _):
                ws = plsc.load_gather(w2b, [jnp.full((16,), j, i32)])
                for t in range(8):
                    r = rowsa[j, pl.ds(t * 16, 16)]
                    rowsb[j, pl.ds(t * 16, 16)] = r * ws
                return 0
            lax.fori_loop(0, 128, _scale, 0)
            pltpu.sync_copy(rowsb, acc_sh.at[idx2.at[0]], add=True)
            return 0
        lax.fori_loop(0, EROWS, _chunk, 0)
        plsc.subcore_barrier()

        for off, sz in ((0, 128), (128, 128), (256, 64)):
            pltpu.sync_copy(acc_sh.at[pl.ds(s * 320 + off, sz), :],
                            rowsa.at[pl.ds(0, sz), :])
            pltpu.sync_copy(
                rowsa.at[pl.ds(0, sz), :],
                acc_out.at[c, pl.ds(lo + s * 320 + off, sz), :])
        plsc.subcore_barrier()


_sce = functools.partial(
    pl.kernel,
    out_type=jax.ShapeDtypeStruct((2, NA, 128), f32),
    mesh=_sc_mesh,
    compiler_params=_sc_params,
    scratch_types=[
        pltpu.VMEM((EROWS, 128), i32),
        pltpu.VMEM((EROWS, 128), i32),
        pltpu.VMEM((NA,), f32),
        pltpu.VMEM((NA,), f32),
        pltpu.VMEM((128, 128), f32),
        pltpu.VMEM((128, 128), f32),
        pltpu.VMEM((128,), f32),
        pltpu.VMEM((1, 128), i32),
        pltpu.VMEM_SHARED((NA // 2, 128), f32),
    ],
)(_sce_body)


def _tcf_body(acc_ref, out_ref):
    a = acc_ref[0] + acc_ref[1]
    out_ref[...] = a[:, :NCLS] / (a[:, 47:48] + EPS)


def _tcf(acc2p):
    return pl.pallas_call(
        _tcf_body,
        grid=(N // NB,),
        in_specs=[pl.BlockSpec((2, NB, 128), lambda i: (0, i, 0))],
        out_specs=pl.BlockSpec((NB, NCLS), lambda i: (i, 0)),
        out_shape=jax.ShapeDtypeStruct((N, NCLS), f32),
    )(acc2p)


_USE_SC_C = True
_USE_SC_E = True


def kernel(x, edge_index, W1, al1, ar1, W2, al2, ar2):
    src = edge_index[0]
    dst = edge_index[1]
    pad = E_PAD - E
    srcp = jnp.concatenate([src, jnp.zeros((pad,), i32)])
    dstp = jnp.concatenate([dst, jnp.zeros((pad,), i32)])
    src2 = srcp.reshape(E_PAD // 128, 128)
    dst2 = dstp.reshape(E_PAD // 128, 128)

    d_big = _build_dbig(al1, ar1)
    W2p = jnp.zeros((HEADS * HID, 128), f32).at[:, :NCLS].set(W2)
    al2p = jnp.zeros((128,), f32).at[:NCLS].set(al2[0])
    ar2p = jnp.zeros((128,), f32).at[:NCLS].set(ar2[0])
    Jmat = jnp.zeros((128, 8), f32).at[:, 0].set(al2p).at[:, 1].set(ar2p)

    feat1_g, T1 = _tca(x, W1, d_big)
    feat1_flat = feat1_g.reshape(NPIECE * N, 128)

    w1e = _scb(T1, src2, dst2)

    out1p, den1p = _scc(feat1_flat, w1e, src2, dst2)

    feat2p, elr2 = _tcd(out1p, den1p, W2p, Jmat)
    el2 = jnp.zeros((NA,), f32).at[:N].set(elr2[:, 0])
    er2 = jnp.zeros((NA,), f32).at[:N].set(elr2[:, 1])

    acc2p = _sce(src2, dst2, feat2p, el2, er2)

    return _tcf(acc2p)


# speedup vs baseline: 4.1758x; 1.0040x over previous
import functools

import jax
import jax.numpy as jnp
from jax import lax
from jax.experimental import pallas as pl
from jax.experimental.pallas import tpu as pltpu
from jax.experimental.pallas import tpu_sc as plsc

N = 10000
E = 160000
F_IN = 256
HID = 256
HEADS = 8
NCLS = 40
NPIECE = 16
E_PAD = 163840
NB = 400
EPS = 1e-30

NW = 32
EW = E_PAD // NW
EROWS = EW // 128
NA = 10240
NROWT = NA // 16
ETILE = E_PAD // 16
ESROWS = 16

f32 = jnp.float32
i32 = jnp.int32

_sc_mesh = plsc.VectorSubcoreMesh(core_axis_name="c", subcore_axis_name="s")
_sc_params = pltpu.CompilerParams(needs_layout_passes=False)


def _build_dbig(al1, ar1):
    H, F = al1.shape
    d = jnp.zeros((H * F, 128), f32)
    for h in range(H):
        for r in (2 * h, 2 * h + 1):
            d = d.at[h * F:(h + 1) * F, r].set(al1[h])
            d = d.at[h * F:(h + 1) * F, 16 + r].set(ar1[h])
    return d


def _tca_body(x_ref, w1_ref, dbig_ref, fg_ref, t_ref):
    f = jnp.dot(x_ref[...], w1_ref[...], preferred_element_type=f32)
    fg_ref[...] = f.reshape(NB, NPIECE, 128).transpose(1, 0, 2)
    t_ref[...] = jnp.dot(f, dbig_ref[...], preferred_element_type=f32)


def _tca(x, W1, d_big):
    return pl.pallas_call(
        _tca_body,
        grid=(N // NB,),
        in_specs=[
            pl.BlockSpec((NB, F_IN), lambda i: (i, 0)),
            pl.BlockSpec((F_IN, HEADS * HID), lambda i: (0, 0)),
            pl.BlockSpec((HEADS * HID, 128), lambda i: (0, 0)),
        ],
        out_specs=[
            pl.BlockSpec((NPIECE, NB, 128), lambda i: (0, i, 0)),
            pl.BlockSpec((NB, 128), lambda i: (i, 0)),
        ],
        out_shape=[
            jax.ShapeDtypeStruct((NPIECE, N, 128), f32),
            jax.ShapeDtypeStruct((N, 128), f32),
        ],
    )(x, W1, d_big)


def _scb_body(t_hbm, src2_hbm, dst2_hbm, w1e_out,
              idx_s, idx_d, elrows, errows, wbuf16, sem_g0, sem_g1, sem_s):
    c = lax.axis_index("c")
    s = lax.axis_index("s")
    wid = s * 2 + c
    rowbase = wid * EROWS
    ebase = wid * EW
    pltpu.sync_copy(src2_hbm.at[pl.ds(rowbase, EROWS), :], idx_s)
    pltpu.sync_copy(dst2_hbm.at[pl.ds(rowbase, EROWS), :], idx_d)

    def _chunk(k, _):
        c0 = pltpu.async_copy(t_hbm.at[idx_s.at[k]], elrows, sem_g0)
        c1 = pltpu.async_copy(t_hbm.at[idx_d.at[k]], errows, sem_g1)

        @pl.when(k > 0)
        def _():
            pltpu.make_async_copy(
                t_hbm.at[pl.ds(0, 128), :], wbuf16, sem_s).wait()

        c0.wait()
        c1.wait()

        def _edge(j, _):
            el = elrows[j, pl.ds(0, 16)]
            er = errows[j, pl.ds(16, 16)]
            e = el + er
            e = jnp.where(e > 0, e, 0.2 * e)
            w = jnp.exp(e)
            eid = ebase + k * 128 + j
            w = jnp.where(jnp.full((16,), eid, i32) < E, w, 0.0)
            wbuf16[j, :] = w
            return 0
        lax.fori_loop(0, 128, _edge, 0, unroll=2)
        pltpu.async_copy(wbuf16, w1e_out.at[pl.ds(ebase + k * 128, 128), :],
                         sem_s)
        return 0
    lax.fori_loop(0, EROWS, _chunk, 0)
    pltpu.make_async_copy(t_hbm.at[pl.ds(0, 128), :], wbuf16, sem_s).wait()


_scb = functools.partial(
    pl.kernel,
    out_type=jax.ShapeDtypeStruct((E_PAD, NPIECE), f32),
    mesh=_sc_mesh,
    compiler_params=_sc_params,
    scratch_types=[
        pltpu.VMEM((EROWS, 128), i32),
        pltpu.VMEM((EROWS, 128), i32),
        pltpu.VMEM((128, 128), f32),
        pltpu.VMEM((128, 128), f32),
        pltpu.VMEM((128, NPIECE), f32),
        pltpu.SemaphoreType.DMA,
        pltpu.SemaphoreType.DMA,
        pltpu.SemaphoreType.DMA,
    ],
)(_scb_body)


def _scc_body(ff_hbm, w1e_hbm, src2_hbm, dst2_hbm, out1p, den1p,
              idx_s, idx_d, qbuf, rowsa0, rowsa1, rowsb, wchunk,
              sem_w, sem_g0, sem_g1, sem_s, acc_sh):
    c = lax.axis_index("c")
    s = lax.axis_index("s")

    def _zero_acc():
        def _zrow(j, _):
            for t in range(8):
                rowsb[j, pl.ds(t * 16, 16)] = jnp.zeros((16,), f32)
            return 0
        lax.fori_loop(0, 128, _zrow, 0)
        for i in range(5):
            r0 = s * NROWT + i * 128
            pltpu.sync_copy(rowsb, acc_sh.at[pl.ds(r0, 128), :])

    def _copy_out(dst):
        for i in range(5):
            r0 = s * NROWT + i * 128
            pltpu.sync_copy(acc_sh.at[pl.ds(r0, 128), :], rowsb)
            pltpu.sync_copy(rowsb, dst.at[pl.ds(r0, 128), :])

    def _pass(p, _):
        q = c * 8 + p
        _zero_acc()
        plsc.subcore_barrier()

        def _stage(stage, _):
            @pl.when(stage > 0)
            def _():
                pltpu.make_async_copy(
                    w1e_hbm.at[pl.ds(0, 128), :], rowsb, sem_s).wait()

            rb = s * 80 + stage * 8
            pltpu.sync_copy(src2_hbm.at[pl.ds(rb, 8), :], idx_s)
            pltpu.sync_copy(dst2_hbm.at[pl.ds(rb, 8), :], idx_d)

            def _chunk(k, _):
                for t in range(8):
                    qbuf[pl.ds(t * 16, 16)] = (
                        idx_s[k, pl.ds(t * 16, 16)] + q * N)
                cw = pltpu.async_copy(
                    w1e_hbm.at[pl.ds(rb * 128 + k * 128, 128), :], wchunk,
                    sem_w)
                ras = (rowsa0, rowsa1)
                sems = (sem_g0, sem_g1)
                cps = [pltpu.async_copy(
                    ff_hbm.at[qbuf.at[pl.ds(0, 32)]], rowsa0, sem_g0)]

                @pl.when(k > 0)
                def _():
                    pltpu.make_async_copy(
                        w1e_hbm.at[pl.ds(0, 128), :], rowsb, sem_s).wait()

                cw.wait()
                for i in range(4):
                    if i < 3:
                        cps.append(pltpu.async_copy(
                            ff_hbm.at[qbuf.at[pl.ds(32 * (i + 1), 32)]],
                            ras[(i + 1) % 2], sems[(i + 1) % 2]))
                    cps[i].wait()
                    ra = ras[i % 2]

                    def _edge(j, _):
                        ws = plsc.load_gather(
                            wchunk,
                            [jnp.full((16,), 32 * i, i32) + j,
                             jnp.full((16,), q, i32)])
                        for t in range(8):
                            r = ra[j, pl.ds(t * 16, 16)]
                            rowsb[32 * i + j, pl.ds(t * 16, 16)] = r * ws
                        return 0
                    lax.fori_loop(0, 32, _edge, 0, unroll=2)
                pltpu.async_copy(rowsb, acc_sh.at[idx_d.at[k]], sem_s,
                                 add=True)
                return 0
            lax.fori_loop(0, 8, _chunk, 0)
            return 0
        lax.fori_loop(0, 10, _stage, 0)
        pltpu.make_async_copy(
            w1e_hbm.at[pl.ds(0, 128), :], rowsb, sem_s).wait()
        plsc.subcore_barrier()
        _copy_out(out1p.at[q])
        plsc.subcore_barrier()
        return 0
    lax.fori_loop(0, 8, _pass, 0)

    _zero_acc()
    plsc.subcore_barrier()

    def _dstage(stage, _):
        rb = c * 640 + s * 40 + stage * 8
        pltpu.sync_copy(dst2_hbm.at[pl.ds(rb, 8), :], idx_d)

        def _dchunk(k, _):
            pltpu.sync_copy(
                w1e_hbm.at[pl.ds(rb * 128 + k * 128, 128), :], wchunk)

            def _edge(j, _):
                rowsb[j, pl.ds(0, 16)] = wchunk[j, :]
                return 0
            lax.fori_loop(0, 128, _edge, 0)
            pltpu.sync_copy(rowsb, acc_sh.at[idx_d.at[k]], add=True)
            return 0
        lax.fori_loop(0, 8, _dchunk, 0)
        return 0
    lax.fori_loop(0, 5, _dstage, 0)
    plsc.subcore_barrier()
    _copy_out(den1p.at[c])


_scc = functools.partial(
    pl.kernel,
    out_type=[
        jax.ShapeDtypeStruct((NPIECE, NA, 128), f32),
        jax.ShapeDtypeStruct((2, NA, 128), f32),
    ],
    mesh=_sc_mesh,
    compiler_params=_sc_params,
    scratch_types=[
        pltpu.VMEM((8, 128), i32),
        pltpu.VMEM((8, 128), i32),
        pltpu.VMEM((128,), i32),
        pltpu.VMEM((32, 128), f32),
        pltpu.VMEM((32, 128), f32),
        pltpu.VMEM((128, 128), f32),
        pltpu.VMEM((128, NPIECE), f32),
        pltpu.SemaphoreType.DMA,
        pltpu.SemaphoreType.DMA,
        pltpu.SemaphoreType.DMA,
        pltpu.SemaphoreType.DMA,
        pltpu.VMEM_SHARED((NA, 128), f32),
    ],
)(_scc_body)


def _tcd_body(og_ref, den_ref, w2_ref, j_ref, f2_ref, elr_ref):
    den = den_ref[0, :, :NPIECE] + den_ref[1, :, :NPIECE] + EPS
    h = og_ref[...].transpose(1, 0, 2) / den[:, :, None]
    h = jax.nn.relu(h.reshape(NB, HEADS * HID))
    f2 = jnp.dot(h, w2_ref[...], preferred_element_type=f32)
    ones_col = lax.broadcasted_iota(i32, (NB, 128), 1) == 47
    f2 = jnp.where(ones_col, 1.0, f2)
    f2_ref[...] = f2
    elr_ref[...] = jnp.dot(f2, j_ref[...], preferred_element_type=f32)


def _tcd(out1p, den1p, W2p, Jmat):
    return pl.pallas_call(
        _tcd_body,
        grid=(N // NB,),
        in_specs=[
            pl.BlockSpec((NPIECE, NB, 128), lambda i: (0, i, 0)),
            pl.BlockSpec((2, NB, 128), lambda i: (0, i, 0)),
            pl.BlockSpec((HEADS * HID, 128), lambda i: (0, 0)),
            pl.BlockSpec((128, 8), lambda i: (0, 0)),
        ],
        out_specs=[
            pl.BlockSpec((NB, 128), lambda i: (i, 0)),
            pl.BlockSpec((NB, 8), lambda i: (i, 0)),
        ],
        out_shape=[
            jax.ShapeDtypeStruct((N, 128), f32),
            jax.ShapeDtypeStruct((N, 8), f32),
        ],
    )(out1p, den1p, W2p, Jmat)


def _sce_body(src2_hbm, dst2_hbm, f2p_hbm, el2_hbm, er2_hbm, acc_out,
              idx_s, idx_d, el2v, er2v, rowsa, rowsb, w2b, idx2,
              sem_g0, sem_g1, sem_s, acc_sh):
    c = lax.axis_index("c")
    s = lax.axis_index("s")
    wid = s * 2 + c
    rowbase = wid * EROWS
    ebase = wid * EW
    pltpu.sync_copy(src2_hbm.at[pl.ds(rowbase, EROWS), :], idx_s)
    pltpu.sync_copy(dst2_hbm.at[pl.ds(rowbase, EROWS), :], idx_d)
    pltpu.sync_copy(el2_hbm, el2v)
    pltpu.sync_copy(er2_hbm, er2v)

    for h in range(2):
        lo = h * (NA // 2)

        def _zrow(j, _):
            for t in range(8):
                rowsb[j, pl.ds(t * 16, 16)] = jnp.zeros((16,), f32)
            return 0
        lax.fori_loop(0, 128, _zrow, 0)
        for off, sz in ((0, 128), (128, 128), (256, 64)):
            pltpu.sync_copy(rowsb.at[pl.ds(0, sz), :],
                            acc_sh.at[pl.ds(s * 320 + off, sz), :])
        plsc.subcore_barrier()

        def _chunk(k, _):
            c0 = pltpu.async_copy(f2p_hbm.at[idx_s.at[k]], rowsa, sem_g0)

            @pl.when(k > 0)
            def _():
                pltpu.make_async_copy(
                    f2p_hbm.at[pl.ds(0, 128), :], rowsb, sem_s).wait()

            for t in range(8):
                sv = idx_s[k, pl.ds(t * 16, 16)]
                dv = idx_d[k, pl.ds(t * 16, 16)]
                elv = plsc.load_gather(el2v, [sv])
                erv = plsc.load_gather(er2v, [dv])
                e = elv + erv
                e = jnp.where(e > 0, e, 0.2 * e)
                w = jnp.exp(e)
                eid = ebase + k * 128 + t * 16 + lax.iota(i32, 16)
                w = jnp.where(eid < E, w, 0.0)
                inh = (dv >= lo) & (dv < lo + NA // 2)
                w = jnp.where(inh, w, 0.0)
                w2b[pl.ds(t * 16, 16)] = w
                iv = jnp.clip(dv - lo, 0, NA // 2 - 1)
                idx2[0, pl.ds(t * 16, 16)] = iv

            c0.wait()

            def _scale(j, _):
                ws = plsc.load_gather(w2b, [jnp.full((16,), j, i32)])
                for t in range(8):
                    r = rowsa[j, pl.ds(t * 16, 16)]
                    rowsb[j, pl.ds(t * 16, 16)] = r * ws
                return 0
            lax.fori_loop(0, 128, _scale, 0, unroll=2)
            pltpu.async_copy(rowsb, acc_sh.at[idx2.at[0]], sem_s, add=True)
            return 0
        lax.fori_loop(0, EROWS, _chunk, 0)
        pltpu.make_async_copy(
            f2p_hbm.at[pl.ds(0, 128), :], rowsb, sem_s).wait()
        plsc.subcore_barrier()

        for off, sz in ((0, 128), (128, 128), (256, 64)):
            pltpu.sync_copy(acc_sh.at[pl.ds(s * 320 + off, sz), :],
                            rowsa.at[pl.ds(0, sz), :])
            pltpu.sync_copy(
                rowsa.at[pl.ds(0, sz), :],
                acc_out.at[c, pl.ds(lo + s * 320 + off, sz), :])
        plsc.subcore_barrier()


_sce = functools.partial(
    pl.kernel,
    out_type=jax.ShapeDtypeStruct((2, NA, 128), f32),
    mesh=_sc_mesh,
    compiler_params=_sc_params,
    scratch_types=[
        pltpu.VMEM((EROWS, 128), i32),
        pltpu.VMEM((EROWS, 128), i32),
        pltpu.VMEM((NA,), f32),
        pltpu.VMEM((NA,), f32),
        pltpu.VMEM((128, 128), f32),
        pltpu.VMEM((128, 128), f32),
        pltpu.VMEM((128,), f32),
        pltpu.VMEM((1, 128), i32),
        pltpu.SemaphoreType.DMA,
        pltpu.SemaphoreType.DMA,
        pltpu.SemaphoreType.DMA,
        pltpu.VMEM_SHARED((NA // 2, 128), f32),
    ],
)(_sce_body)


def _tcf_body(acc_ref, out_ref):
    a = acc_ref[0] + acc_ref[1]
    out_ref[...] = a[:, :NCLS] / (a[:, 47:48] + EPS)


def _tcf(acc2p):
    return pl.pallas_call(
        _tcf_body,
        grid=(N // NB,),
        in_specs=[pl.BlockSpec((2, NB, 128), lambda i: (0, i, 0))],
        out_specs=pl.BlockSpec((NB, NCLS), lambda i: (i, 0)),
        out_shape=jax.ShapeDtypeStruct((N, NCLS), f32),
    )(acc2p)


_USE_SC_C = True
_USE_SC_E = True


def kernel(x, edge_index, W1, al1, ar1, W2, al2, ar2):
    src = edge_index[0]
    dst = edge_index[1]
    pad = E_PAD - E
    srcp = jnp.concatenate([src, jnp.zeros((pad,), i32)])
    dstp = jnp.concatenate([dst, jnp.zeros((pad,), i32)])
    src2 = srcp.reshape(E_PAD // 128, 128)
    dst2 = dstp.reshape(E_PAD // 128, 128)

    d_big = _build_dbig(al1, ar1)
    W2p = jnp.zeros((HEADS * HID, 128), f32).at[:, :NCLS].set(W2)
    al2p = jnp.zeros((128,), f32).at[:NCLS].set(al2[0])
    ar2p = jnp.zeros((128,), f32).at[:NCLS].set(ar2[0])
    Jmat = jnp.zeros((128, 8), f32).at[:, 0].set(al2p).at[:, 1].set(ar2p)

    feat1_g, T1 = _tca(x, W1, d_big)
    feat1_flat = feat1_g.reshape(NPIECE * N, 128)

    w1e = _scb(T1, src2, dst2)

    out1p, den1p = _scc(feat1_flat, w1e, src2, dst2)

    feat2p, elr2 = _tcd(out1p, den1p, W2p, Jmat)
    el2 = jnp.zeros((NA,), f32).at[:N].set(elr2[:, 0])
    er2 = jnp.zeros((NA,), f32).at[:N].set(elr2[:, 1])

    acc2p = _sce(src2, dst2, feat2p, el2, er2)

    return _tcf(acc2p)


# speedup vs baseline: 4.2270x; 1.0122x over previous
import functools

import jax
import jax.numpy as jnp
from jax import lax
from jax.experimental import pallas as pl
from jax.experimental.pallas import tpu as pltpu
from jax.experimental.pallas import tpu_sc as plsc

N = 10000
E = 160000
F_IN = 256
HID = 256
HEADS = 8
NCLS = 40
NPIECE = 16
E_PAD = 163840
NB = 400
EPS = 1e-30

NW = 32
EW = E_PAD // NW
EROWS = EW // 128
NA = 10240
NROWT = NA // 16
ETILE = E_PAD // 16
ESROWS = 16

f32 = jnp.float32
i32 = jnp.int32

_sc_mesh = plsc.VectorSubcoreMesh(core_axis_name="c", subcore_axis_name="s")
_sc_params = pltpu.CompilerParams(needs_layout_passes=False)


def _build_dbig(al1, ar1):
    H, F = al1.shape
    d = jnp.zeros((H * F, 128), f32)
    for h in range(H):
        for r in (2 * h, 2 * h + 1):
            d = d.at[h * F:(h + 1) * F, r].set(al1[h])
            d = d.at[h * F:(h + 1) * F, 16 + r].set(ar1[h])
    return d


def _tca_body(x_ref, w1_ref, dbig_ref, fg_ref, t_ref):
    f = jnp.dot(x_ref[...], w1_ref[...], preferred_element_type=f32)
    fg_ref[...] = f.reshape(NB, NPIECE, 128).transpose(1, 0, 2)
    t_ref[...] = jnp.dot(f, dbig_ref[...], preferred_element_type=f32)


def _tca(x, W1, d_big):
    return pl.pallas_call(
        _tca_body,
        grid=(N // NB,),
        in_specs=[
            pl.BlockSpec((NB, F_IN), lambda i: (i, 0)),
            pl.BlockSpec((F_IN, HEADS * HID), lambda i: (0, 0)),
            pl.BlockSpec((HEADS * HID, 128), lambda i: (0, 0)),
        ],
        out_specs=[
            pl.BlockSpec((NPIECE, NB, 128), lambda i: (0, i, 0)),
            pl.BlockSpec((NB, 128), lambda i: (i, 0)),
        ],
        out_shape=[
            jax.ShapeDtypeStruct((NPIECE, N, 128), f32),
            jax.ShapeDtypeStruct((N, 128), f32),
        ],
    )(x, W1, d_big)


def _scb_body(t_hbm, src2_hbm, dst2_hbm, w1e_out,
              idx_s, idx_d, elrows, errows, wbuf16, sem_g0, sem_g1, sem_s):
    c = lax.axis_index("c")
    s = lax.axis_index("s")
    wid = s * 2 + c
    rowbase = wid * EROWS
    ebase = wid * EW
    pltpu.sync_copy(src2_hbm.at[pl.ds(rowbase, EROWS), :], idx_s)
    pltpu.sync_copy(dst2_hbm.at[pl.ds(rowbase, EROWS), :], idx_d)

    def _chunk(k, _):
        c0 = pltpu.async_copy(t_hbm.at[idx_s.at[k]], elrows, sem_g0)
        c1 = pltpu.async_copy(t_hbm.at[idx_d.at[k]], errows, sem_g1)

        @pl.when(k > 0)
        def _():
            pltpu.make_async_copy(
                t_hbm.at[pl.ds(0, 128), :], wbuf16, sem_s).wait()

        c0.wait()
        c1.wait()

        def _edge(j, _):
            el = elrows[j, pl.ds(0, 16)]
            er = errows[j, pl.ds(16, 16)]
            e = el + er
            e = jnp.where(e > 0, e, 0.2 * e)
            w = jnp.exp(e)
            eid = ebase + k * 128 + j
            w = jnp.where(jnp.full((16,), eid, i32) < E, w, 0.0)
            wbuf16[j, :] = w
            return 0
        lax.fori_loop(0, 128, _edge, 0, unroll=2)
        pltpu.async_copy(wbuf16, w1e_out.at[pl.ds(ebase + k * 128, 128), :],
                         sem_s)
        return 0
    lax.fori_loop(0, EROWS, _chunk, 0)
    pltpu.make_async_copy(t_hbm.at[pl.ds(0, 128), :], wbuf16, sem_s).wait()


_scb = functools.partial(
    pl.kernel,
    out_type=jax.ShapeDtypeStruct((E_PAD, NPIECE), f32),
    mesh=_sc_mesh,
    compiler_params=_sc_params,
    scratch_types=[
        pltpu.VMEM((EROWS, 128), i32),
        pltpu.VMEM((EROWS, 128), i32),
        pltpu.VMEM((128, 128), f32),
        pltpu.VMEM((128, 128), f32),
        pltpu.VMEM((128, NPIECE), f32),
        pltpu.SemaphoreType.DMA,
        pltpu.SemaphoreType.DMA,
        pltpu.SemaphoreType.DMA,
    ],
)(_scb_body)


def _scc_body(ff_hbm, w1e_hbm, src2_hbm, dst2_hbm, out1p, den1p,
              idx_s, idx_d, qbuf, rowsa0, rowsa1, rowsa2, rowsa3, rowsb,
              wchunk, sem_w, sem_g0, sem_g1, sem_g2, sem_g3, sem_s, acc_sh):
    c = lax.axis_index("c")
    s = lax.axis_index("s")

    def _zero_acc():
        def _zrow(j, _):
            for t in range(8):
                rowsb[j, pl.ds(t * 16, 16)] = jnp.zeros((16,), f32)
            return 0
        lax.fori_loop(0, 128, _zrow, 0)
        for i in range(5):
            r0 = s * NROWT + i * 128
            pltpu.sync_copy(rowsb, acc_sh.at[pl.ds(r0, 128), :])

    def _copy_out(dst):
        for i in range(5):
            r0 = s * NROWT + i * 128
            pltpu.sync_copy(acc_sh.at[pl.ds(r0, 128), :], rowsb)
            pltpu.sync_copy(rowsb, dst.at[pl.ds(r0, 128), :])

    def _pass(p, _):
        q = c * 8 + p
        _zero_acc()
        plsc.subcore_barrier()

        def _stage(stage, _):
            @pl.when(stage > 0)
            def _():
                pltpu.make_async_copy(
                    w1e_hbm.at[pl.ds(0, 128), :], rowsb, sem_s).wait()

            rb = s * 80 + stage * 8
            pltpu.sync_copy(src2_hbm.at[pl.ds(rb, 8), :], idx_s)
            pltpu.sync_copy(dst2_hbm.at[pl.ds(rb, 8), :], idx_d)

            def _chunk(k, _):
                for t in range(8):
                    qbuf[pl.ds(t * 16, 16)] = (
                        idx_s[k, pl.ds(t * 16, 16)] + q * N)
                cw = pltpu.async_copy(
                    w1e_hbm.at[pl.ds(rb * 128 + k * 128, 128), :], wchunk,
                    sem_w)
                ras = (rowsa0, rowsa1, rowsa2, rowsa3)
                sems = (sem_g0, sem_g1, sem_g2, sem_g3)
                cps = [pltpu.async_copy(
                    ff_hbm.at[qbuf.at[pl.ds(16 * i, 16)]], ras[i], sems[i])
                    for i in range(4)]

                @pl.when(k > 0)
                def _():
                    pltpu.make_async_copy(
                        w1e_hbm.at[pl.ds(0, 128), :], rowsb, sem_s).wait()

                cw.wait()
                for i in range(8):
                    cps[i].wait()
                    ra = ras[i % 4]

                    def _edge(j, _):
                        ws = plsc.load_gather(
                            wchunk,
                            [jnp.full((16,), 16 * i, i32) + j,
                             jnp.full((16,), q, i32)])
                        for t in range(8):
                            r = ra[j, pl.ds(t * 16, 16)]
                            rowsb[16 * i + j, pl.ds(t * 16, 16)] = r * ws
                        return 0
                    lax.fori_loop(0, 16, _edge, 0, unroll=2)
                    if i + 4 < 8:
                        cps.append(pltpu.async_copy(
                            ff_hbm.at[qbuf.at[pl.ds(16 * (i + 4), 16)]],
                            ras[i % 4], sems[i % 4]))
                pltpu.async_copy(rowsb, acc_sh.at[idx_d.at[k]], sem_s,
                                 add=True)
                return 0
            lax.fori_loop(0, 8, _chunk, 0)
            return 0
        lax.fori_loop(0, 10, _stage, 0)
        pltpu.make_async_copy(
            w1e_hbm.at[pl.ds(0, 128), :], rowsb, sem_s).wait()
        plsc.subcore_barrier()
        _copy_out(out1p.at[q])
        plsc.subcore_barrier()
        return 0
    lax.fori_loop(0, 8, _pass, 0)

    _zero_acc()
    plsc.subcore_barrier()

    def _dstage(stage, _):
        rb = c * 640 + s * 40 + stage * 8
        pltpu.sync_copy(dst2_hbm.at[pl.ds(rb, 8), :], idx_d)

        def _dchunk(k, _):
            pltpu.sync_copy(
                w1e_hbm.at[pl.ds(rb * 128 + k * 128, 128), :], wchunk)

            def _edge(j, _):
                rowsb[j, pl.ds(0, 16)] = wchunk[j, :]
                return 0
            lax.fori_loop(0, 128, _edge, 0)
            pltpu.sync_copy(rowsb, acc_sh.at[idx_d.at[k]], add=True)
            return 0
        lax.fori_loop(0, 8, _dchunk, 0)
        return 0
    lax.fori_loop(0, 5, _dstage, 0)
    plsc.subcore_barrier()
    _copy_out(den1p.at[c])


_scc = functools.partial(
    pl.kernel,
    out_type=[
        jax.ShapeDtypeStruct((NPIECE, NA, 128), f32),
        jax.ShapeDtypeStruct((2, NA, 128), f32),
    ],
    mesh=_sc_mesh,
    compiler_params=_sc_params,
    scratch_types=[
        pltpu.VMEM((8, 128), i32),
        pltpu.VMEM((8, 128), i32),
        pltpu.VMEM((128,), i32),
        pltpu.VMEM((16, 128), f32),
        pltpu.VMEM((16, 128), f32),
        pltpu.VMEM((16, 128), f32),
        pltpu.VMEM((16, 128), f32),
        pltpu.VMEM((128, 128), f32),
        pltpu.VMEM((128, NPIECE), f32),
        pltpu.SemaphoreType.DMA,
        pltpu.SemaphoreType.DMA,
        pltpu.SemaphoreType.DMA,
        pltpu.SemaphoreType.DMA,
        pltpu.SemaphoreType.DMA,
        pltpu.SemaphoreType.DMA,
        pltpu.VMEM_SHARED((NA, 128), f32),
    ],
)(_scc_body)


def _tcd_body(og_ref, den_ref, w2_ref, j_ref, f2_ref, elr_ref):
    den = den_ref[0, :, :NPIECE] + den_ref[1, :, :NPIECE] + EPS
    h = og_ref[...].transpose(1, 0, 2) / den[:, :, None]
    h = jax.nn.relu(h.reshape(NB, HEADS * HID))
    f2 = jnp.dot(h, w2_ref[...], preferred_element_type=f32)
    ones_col = lax.broadcasted_iota(i32, (NB, 128), 1) == 47
    f2 = jnp.where(ones_col, 1.0, f2)
    f2_ref[...] = f2
    elr_ref[...] = jnp.dot(f2, j_ref[...], preferred_element_type=f32)


def _tcd(out1p, den1p, W2p, Jmat):
    return pl.pallas_call(
        _tcd_body,
        grid=(N // NB,),
        in_specs=[
            pl.BlockSpec((NPIECE, NB, 128), lambda i: (0, i, 0)),
            pl.BlockSpec((2, NB, 128), lambda i: (0, i, 0)),
            pl.BlockSpec((HEADS * HID, 128), lambda i: (0, 0)),
            pl.BlockSpec((128, 8), lambda i: (0, 0)),
        ],
        out_specs=[
            pl.BlockSpec((NB, 128), lambda i: (i, 0)),
            pl.BlockSpec((NB, 8), lambda i: (i, 0)),
        ],
        out_shape=[
            jax.ShapeDtypeStruct((N, 128), f32),
            jax.ShapeDtypeStruct((N, 8), f32),
        ],
    )(out1p, den1p, W2p, Jmat)


def _sce_body(src2_hbm, dst2_hbm, f2p_hbm, el2_hbm, er2_hbm, acc_out,
              idx_s, idx_d, el2v, er2v, rowsa, rowsb, w2b, idx2,
              sem_g0, sem_g1, sem_s, acc_sh):
    c = lax.axis_index("c")
    s = lax.axis_index("s")
    wid = s * 2 + c
    rowbase = wid * EROWS
    ebase = wid * EW
    pltpu.sync_copy(src2_hbm.at[pl.ds(rowbase, EROWS), :], idx_s)
    pltpu.sync_copy(dst2_hbm.at[pl.ds(rowbase, EROWS), :], idx_d)
    pltpu.sync_copy(el2_hbm, el2v)
    pltpu.sync_copy(er2_hbm, er2v)

    for h in range(2):
        lo = h * (NA // 2)

        def _zrow(j, _):
            for t in range(8):
                rowsb[j, pl.ds(t * 16, 16)] = jnp.zeros((16,), f32)
            return 0
        lax.fori_loop(0, 128, _zrow, 0)
        for off, sz in ((0, 128), (128, 128), (256, 64)):
            pltpu.sync_copy(rowsb.at[pl.ds(0, sz), :],
                            acc_sh.at[pl.ds(s * 320 + off, sz), :])
        plsc.subcore_barrier()

        def _chunk(k, _):
            c0 = pltpu.async_copy(f2p_hbm.at[idx_s.at[k]], rowsa, sem_g0)

            @pl.when(k > 0)
            def _():
                pltpu.make_async_copy(
                    f2p_hbm.at[pl.ds(0, 128), :], rowsb, sem_s).wait()

            for t in range(8):
                sv = idx_s[k, pl.ds(t * 16, 16)]
                dv = idx_d[k, pl.ds(t * 16, 16)]
                elv = plsc.load_gather(el2v, [sv])
                erv = plsc.load_gather(er2v, [dv])
                e = elv + erv
                e = jnp.where(e > 0, e, 0.2 * e)
                w = jnp.exp(e)
                eid = ebase + k * 128 + t * 16 + lax.iota(i32, 16)
                w = jnp.where(eid < E, w, 0.0)
                inh = (dv >= lo) & (dv < lo + NA // 2)
                w = jnp.where(inh, w, 0.0)
                w2b[pl.ds(t * 16, 16)] = w
                iv = jnp.clip(dv - lo, 0, NA // 2 - 1)
                idx2[0, pl.ds(t * 16, 16)] = iv

            c0.wait()

            def _scale(j, _):
                ws = plsc.load_gather(w2b, [jnp.full((16,), j, i32)])
                for t in range(8):
                    r = rowsa[j, pl.ds(t * 16, 16)]
                    rowsb[j, pl.ds(t * 16, 16)] = r * ws
                return 0
            lax.fori_loop(0, 128, _scale, 0, unroll=2)
            pltpu.async_copy(rowsb, acc_sh.at[idx2.at[0]], sem_s, add=True)
            return 0
        lax.fori_loop(0, EROWS, _chunk, 0)
        pltpu.make_async_copy(
            f2p_hbm.at[pl.ds(0, 128), :], rowsb, sem_s).wait()
        plsc.subcore_barrier()

        for off, sz in ((0, 128), (128, 128), (256, 64)):
            pltpu.sync_copy(acc_sh.at[pl.ds(s * 320 + off, sz), :],
                            rowsa.at[pl.ds(0, sz), :])
            pltpu.sync_copy(
                rowsa.at[pl.ds(0, sz), :],
                acc_out.at[c, pl.ds(lo + s * 320 + off, sz), :])
        plsc.subcore_barrier()


_sce = functools.partial(
    pl.kernel,
    out_type=jax.ShapeDtypeStruct((2, NA, 128), f32),
    mesh=_sc_mesh,
    compiler_params=_sc_params,
    scratch_types=[
        pltpu.VMEM((EROWS, 128), i32),
        pltpu.VMEM((EROWS, 128), i32),
        pltpu.VMEM((NA,), f32),
        pltpu.VMEM((NA,), f32),
        pltpu.VMEM((128, 128), f32),
        pltpu.VMEM((128, 128), f32),
        pltpu.VMEM((128,), f32),
        pltpu.VMEM((1, 128), i32),
        pltpu.SemaphoreType.DMA,
        pltpu.SemaphoreType.DMA,
        pltpu.SemaphoreType.DMA,
        pltpu.VMEM_SHARED((NA // 2, 128), f32),
    ],
)(_sce_body)


def _tcf_body(acc_ref, out_ref):
    a = acc_ref[0] + acc_ref[1]
    out_ref[...] = a[:, :NCLS] / (a[:, 47:48] + EPS)


def _tcf(acc2p):
    return pl.pallas_call(
        _tcf_body,
        grid=(N // NB,),
        in_specs=[pl.BlockSpec((2, NB, 128), lambda i: (0, i, 0))],
        out_specs=pl.BlockSpec((NB, NCLS), lambda i: (i, 0)),
        out_shape=jax.ShapeDtypeStruct((N, NCLS), f32),
    )(acc2p)


_USE_SC_C = True
_USE_SC_E = True


def kernel(x, edge_index, W1, al1, ar1, W2, al2, ar2):
    src = edge_index[0]
    dst = edge_index[1]
    pad = E_PAD - E
    srcp = jnp.concatenate([src, jnp.zeros((pad,), i32)])
    dstp = jnp.concatenate([dst, jnp.zeros((pad,), i32)])
    src2 = srcp.reshape(E_PAD // 128, 128)
    dst2 = dstp.reshape(E_PAD // 128, 128)

    d_big = _build_dbig(al1, ar1)
    W2p = jnp.zeros((HEADS * HID, 128), f32).at[:, :NCLS].set(W2)
    al2p = jnp.zeros((128,), f32).at[:NCLS].set(al2[0])
    ar2p = jnp.zeros((128,), f32).at[:NCLS].set(ar2[0])
    Jmat = jnp.zeros((128, 8), f32).at[:, 0].set(al2p).at[:, 1].set(ar2p)

    feat1_g, T1 = _tca(x, W1, d_big)
    feat1_flat = feat1_g.reshape(NPIECE * N, 128)

    w1e = _scb(T1, src2, dst2)

    out1p, den1p = _scc(feat1_flat, w1e, src2, dst2)

    feat2p, elr2 = _tcd(out1p, den1p, W2p, Jmat)
    el2 = jnp.zeros((NA,), f32).at[:N].set(elr2[:, 0])
    er2 = jnp.zeros((NA,), f32).at[:N].set(elr2[:, 1])

    acc2p = _sce(src2, dst2, feat2p, el2, er2)

    return _tcf(acc2p)
